# baseline (speedup 1.0000x reference)
import jax
import jax.numpy as jnp
from jax import lax
from jax.experimental import pallas as pl
from jax.experimental.pallas import tpu as pltpu

_G = 66
_P = _G * _G
_OUT_L = 4155


def _fused_kernel(pt_ref, w1_ref, b1_ref, w2_ref, b2_ref, o_ref):
    pt = pt_ref[0]

    a1 = jnp.dot(w1_ref[...], pt, preferred_element_type=jnp.float32)
    a1 = a1 + b1_ref[:, 0:1]
    a1 = jnp.where(a1 > 0, a1, 0.2 * a1)

    p = lax.broadcasted_iota(jnp.int32, (1, _P), 1)
    ii = p // _G
    jj = p - ii * _G
    mask = (ii >= 1) & (ii <= 64) & (jj >= 1) & (jj <= 64)
    a1 = jnp.where(mask, a1, 0.0)

    ut = jnp.dot(w2_ref[...], a1, preferred_element_type=jnp.float32)

    acc = b2_ref[0:1, 0:1] + jnp.zeros((1, _OUT_L), jnp.float32)
    for kh in range(4):
        for kw in range(4):
            t = kh * 4 + kw
            off = kh * _G + kw
            acc = acc + ut[t:t + 1, off:off + _OUT_L]
    o_ref[0] = 1.0 / (1.0 + jnp.exp(-acc))


def kernel(x, w1, b1, w2, b2):
    n = x.shape[0]

    xp = jnp.pad(x[:, 0], ((0, 0), (3, 3), (3, 3)))
    taps = [xp[:, kh:kh + 132:2, kw:kw + 132:2]
            for kh in range(4) for kw in range(4)]
    t = jnp.stack(taps, axis=1).astype(jnp.bfloat16)
    pt = t.reshape(n, 16, _P)

    w1c = w1.reshape(64, 16).astype(jnp.bfloat16)
    b1c = jnp.broadcast_to(b1.reshape(64, 1), (64, 128))
    w2c = jnp.transpose(w2.reshape(64, 16))
    b2c = jnp.broadcast_to(b2.reshape(1, 1), (8, 128))

    cost = pl.CostEstimate(
        flops=2 * n * _P * (64 * 16 + 16 * 64) + n * _OUT_L * 20,
        transcendentals=n * _OUT_L,
        bytes_accessed=2 * n * 16 * _P + 4 * (n * _OUT_L + 2 * 64 * 16),
    )
    out = pl.pallas_call(
        _fused_kernel,
        out_shape=jax.ShapeDtypeStruct((n, 1, _OUT_L), jnp.float32),
        grid=(n,),
        in_specs=[
            pl.BlockSpec((1, 16, _P), lambda i: (i, 0, 0)),
            pl.BlockSpec((64, 16), lambda i: (0, 0)),
            pl.BlockSpec((64, 128), lambda i: (0, 0)),
            pl.BlockSpec((16, 64), lambda i: (0, 0)),
            pl.BlockSpec((8, 128), lambda i: (0, 0)),
        ],
        out_specs=pl.BlockSpec((1, 1, _OUT_L), lambda i: (i, 0, 0)),
        compiler_params=pltpu.CompilerParams(
            dimension_semantics=("parallel",)),
        cost_estimate=cost,
    )(pt, w1c, b1c, w2c, b2c)

    o = jnp.pad(out[:, 0], ((0, 0), (0, 63 * _G - _OUT_L)))
    o = o.reshape(n, 63, _G)[:, :, :63]
    return o[:, None]


# speedup vs baseline: 5.6927x; 5.6927x over previous
import jax
import jax.numpy as jnp
from jax import lax
from jax.experimental import pallas as pl
from jax.experimental.pallas import tpu as pltpu

_G = 66
_P = _G * _G
_PL = 4480
_PB_L = 4608
_OUT_L = 4224


def _fused_kernel(pl_ref, w1_ref, b1_ref, w2_ref, b2_ref, o_ref, pb_ref):
    planes = pl_ref[0]

    for u in range(_G):
        pb_ref[0:4, u * _G:u * _G + _G] = planes[:, u, 0:_G]

    pb = pb_ref[0:4, :]
    pt = jnp.concatenate(
        [pb[:, d:d + _PL] for d in (0, 1, _G, _G + 1)], axis=0)

    a1 = jnp.dot(w1_ref[...], pt, preferred_element_type=jnp.float32)
    a1 = a1 + b1_ref[:, 0:1]
    a1 = jnp.where(a1 > 0, a1, 0.2 * a1)

    p = lax.broadcasted_iota(jnp.int32, (1, _PL), 1)
    ii = p // _G
    jj = p - ii * _G
    mask = (ii >= 1) & (ii <= 64) & (jj >= 1) & (jj <= 64)
    a1 = jnp.where(mask, a1, 0.0)

    ut = jnp.dot(w2_ref[...], a1, preferred_element_type=jnp.float32)

    acc = b2_ref[0:1, 0:1] + jnp.zeros((1, _OUT_L), jnp.float32)
    for kh in range(4):
        for kw in range(4):
            t = kh * 4 + kw
            off = kh * _G + kw
            acc = acc + ut[t:t + 1, off:off + _OUT_L]
    o_ref[0] = 1.0 / (1.0 + jnp.exp(-acc))


def kernel(x, w1, b1, w2, b2):
    n = x.shape[0]

    xp = jnp.pad(x[:, 0], ((0, 0), (3, 13), (3, 125)))
    planes = xp.reshape(n, 72, 2, 128, 2)
    planes = planes.transpose(0, 2, 4, 1, 3).reshape(n, 4, 72, 128)

    w1m = w1.reshape(64, 16)
    perm = [(2 * a + p) * 4 + (2 * b + r)
            for a in range(2) for b in range(2)
            for p in range(2) for r in range(2)]
    w1g = w1m[:, jnp.array(perm)]
    b1c = jnp.broadcast_to(b1.reshape(64, 1), (64, 128))
    w2c = jnp.transpose(w2.reshape(64, 16))
    b2c = jnp.broadcast_to(b2.reshape(1, 1), (8, 128))

    cost = pl.CostEstimate(
        flops=2 * n * _PL * (64 * 16 + 16 * 64) + n * _OUT_L * 20,
        transcendentals=n * _OUT_L,
        bytes_accessed=4 * (n * 4 * 72 * 128 + n * _OUT_L + 2 * 64 * 16),
    )
    out = pl.pallas_call(
        _fused_kernel,
        out_shape=jax.ShapeDtypeStruct((n, 1, _OUT_L), jnp.float32),
        grid=(n,),
        in_specs=[
            pl.BlockSpec((1, 4, 72, 128), lambda i: (i, 0, 0, 0)),
            pl.BlockSpec((64, 16), lambda i: (0, 0)),
            pl.BlockSpec((64, 128), lambda i: (0, 0)),
            pl.BlockSpec((16, 64), lambda i: (0, 0)),
            pl.BlockSpec((8, 128), lambda i: (0, 0)),
        ],
        out_specs=pl.BlockSpec((1, 1, _OUT_L), lambda i: (i, 0, 0)),
        scratch_shapes=[pltpu.VMEM((8, _PB_L), jnp.float32)],
        compiler_params=pltpu.CompilerParams(
            dimension_semantics=("parallel",)),
        cost_estimate=cost,
    )(planes, w1g, b1c, w2c, b2c)

    o = out[:, 0, :63 * _G].reshape(n, 63, _G)[:, :, :63]
    return o[:, None]
